# Initial kernel scaffold; baseline (speedup 1.0000x reference)
import jax
import jax.numpy as jnp
from jax.experimental import pallas as pl


def kernel(x, params):
    raise NotImplementedError("write your pallas kernel here")


# speedup vs baseline: 42.3635x; 42.3635x over previous
import functools

import jax
import jax.numpy as jnp
from jax import lax
from jax.experimental import pallas as pl
from jax.experimental.pallas import tpu as pltpu
from jax.experimental.pallas import tpu_sc as plsc

_NUM_BINS = 5
_TB = 2.5
_LN2 = 0.6931471805599453

_NC = 2
_NS = 16
_NW = _NC * _NS
_LANES = 16

_N = 4194304
_PER_W = _N // _NW
_CH = 16384
_CHUNKS = _PER_W // _CH


def _vlog(t):
  bits = plsc.bitcast(t, jnp.int32)
  e = ((bits >> 23) - 127).astype(jnp.float32)
  m = plsc.bitcast((bits & 0x007FFFFF) | 0x3F800000, jnp.float32)
  s = (m - 1.0) / (m + 1.0)
  s2 = s * s
  p = 2.0 / 9.0
  p = 2.0 / 7.0 + s2 * p
  p = 2.0 / 5.0 + s2 * p
  p = 2.0 / 3.0 + s2 * p
  return e * _LN2 + s * (2.0 + s2 * p)


def _lane_shift(v, idx):
  return v.at[idx].get(mode="promise_in_bounds")


def _sc_body(x_hbm, p_hbm, z_hbm, lj_hbm, pbuf, tab, xbuf, zbuf, ljbuf):
  wid = lax.axis_index("s") * _NC + lax.axis_index("c")
  base = wid * _PER_W

  pltpu.sync_copy(p_hbm, pbuf)
  pv = pbuf[...]
  io = lax.iota(jnp.int32, 16)
  mask_w = io < _NUM_BINS
  mask_h = (io >= _NUM_BINS) & (io < 2 * _NUM_BINS)
  neg = jnp.float32(-3.4e38)

  mw = jnp.max(jnp.where(mask_w, pv, neg))
  ew = jnp.exp(pv - mw)
  sw = jnp.sum(jnp.where(mask_w, ew, 0.0))
  w_v = (ew * (2.0 * _TB)) / sw
  mh = jnp.max(jnp.where(mask_h, pv, neg))
  eh = jnp.exp(pv - mh)
  sh = jnp.sum(jnp.where(mask_h, eh, 0.0))
  h_v = (eh * (2.0 * _TB)) / sh
  d_v = jnp.maximum(pv, 0.0) + _vlog(1.0 + jnp.exp(-jnp.abs(pv))) + 1e-5

  cw = plsc.cumsum(jnp.where(mask_w, w_v, 0.0))
  ch = plsc.cumsum(jnp.where(mask_h, h_v, 0.0))

  cap = jnp.int32(15)
  x_k1 = cw - _TB
  x_k = jnp.where(io == 0, -_TB,
                  _lane_shift(cw, jnp.maximum(io - 1, 0)) - _TB)
  rw = 1.0 / (x_k1 - x_k + 1e-8)
  y_k = jnp.where(io == 0, -_TB,
                  _lane_shift(ch, jnp.minimum(io + 4, cap)) - _TB)
  y_k1 = _lane_shift(ch, jnp.minimum(io + 5, cap)) - _TB
  dy = y_k1 - y_k
  d_k = _lane_shift(d_v, jnp.minimum(io + 10, cap))
  d_k1 = _lane_shift(d_v, jnp.minimum(io + 11, cap))
  s_k = _lane_shift(h_v, jnp.minimum(io + 5, cap)) / w_v
  mid = d_k + d_k1 - 2.0 * s_k

  tab[0, :] = x_k
  tab[1, :] = rw
  tab[2, :] = y_k
  tab[3, :] = dy
  tab[4, :] = d_k
  tab[5, :] = d_k1
  tab[6, :] = s_k
  tab[7, :] = mid

  k1 = jnp.sum(jnp.where(io == 0, x_k1, 0.0))
  k2 = jnp.sum(jnp.where(io == 1, x_k1, 0.0))
  k3 = jnp.sum(jnp.where(io == 2, x_k1, 0.0))
  k4 = jnp.sum(jnp.where(io == 3, x_k1, 0.0))

  q_idx = [jnp.full((16,), q, jnp.int32) for q in range(8)]

  def step(i, _):
    sl = pl.ds(i * _LANES, _LANES)
    xv = xbuf[sl]
    inside = (xv >= -_TB) & (xv <= _TB)
    xs = jnp.where(inside, xv, 0.0)
    b = ((k1 < xs).astype(jnp.int32) + (k2 < xs).astype(jnp.int32)
         + (k3 < xs).astype(jnp.int32) + (k4 < xs).astype(jnp.int32))
    g_xk = plsc.load_gather(tab, [q_idx[0], b])
    g_rw = plsc.load_gather(tab, [q_idx[1], b])
    g_yk = plsc.load_gather(tab, [q_idx[2], b])
    g_dy = plsc.load_gather(tab, [q_idx[3], b])
    g_dk = plsc.load_gather(tab, [q_idx[4], b])
    g_dk1 = plsc.load_gather(tab, [q_idx[5], b])
    g_sk = plsc.load_gather(tab, [q_idx[6], b])
    g_mid = plsc.load_gather(tab, [q_idx[7], b])

    xi = jnp.clip((xs - g_xk) * g_rw, 0.0, 1.0)
    om = 1.0 - xi
    t = xi * om
    xi2 = xi * xi
    den = g_sk + g_mid * t
    numz = g_sk * xi2 + g_dk * t
    z_in = g_yk + g_dy * numz / (den + 1e-8)
    numj = g_dk1 * xi2 + 2.0 * (g_sk * t) + g_dk * (om * om) + 1e-8
    r = (g_sk + 1e-8) / (jnp.abs(den) + 1e-8)
    lj_in = _vlog(numj * (r * r))
    zbuf[sl] = jnp.where(inside, z_in, xv)
    ljbuf[sl] = jnp.where(inside, lj_in, 0.0)
    return 0

  def chunk(g, _):
    off = base + g * _CH
    pltpu.sync_copy(x_hbm.at[pl.ds(off, _CH)], xbuf)
    lax.fori_loop(0, _CH // _LANES, step, 0)
    pltpu.sync_copy(zbuf, z_hbm.at[pl.ds(off, _CH)])
    pltpu.sync_copy(ljbuf, lj_hbm.at[pl.ds(off, _CH)])
    return 0

  lax.fori_loop(0, _CHUNKS, chunk, 0)


@jax.jit
def _run(x_flat, params):
  mesh = plsc.VectorSubcoreMesh(core_axis_name="c", subcore_axis_name="s",
                                num_cores=_NC, num_subcores=_NS)
  f = pl.kernel(
      _sc_body,
      out_type=[jax.ShapeDtypeStruct((_N,), jnp.float32),
                jax.ShapeDtypeStruct((_N,), jnp.float32)],
      mesh=mesh,
      compiler_params=pltpu.CompilerParams(needs_layout_passes=False),
      scratch_types=[
          pltpu.VMEM((16,), jnp.float32),
          pltpu.VMEM((8, 16), jnp.float32),
          pltpu.VMEM((_CH,), jnp.float32),
          pltpu.VMEM((_CH,), jnp.float32),
          pltpu.VMEM((_CH,), jnp.float32),
      ],
  )
  return f(x_flat, params)


def kernel(x, params):
  z, lj = _run(x[:, 0], params)
  return (z[:, None], lj)


# speedup vs baseline: 131.0302x; 3.0930x over previous
import functools

import jax
import jax.numpy as jnp
from jax import lax
from jax.experimental import pallas as pl
from jax.experimental.pallas import tpu as pltpu
from jax.experimental.pallas import tpu_sc as plsc

_NUM_BINS = 5
_TB = 2.5
_LN2 = 0.6931471805599453

_NC = 2
_NS = 16
_NW = _NC * _NS
_LANES = 16

_N = 4194304
_PER_W = _N // _NW
_CH = 16384
_CHUNKS = _PER_W // _CH


def _vlog(t):
  bits = plsc.bitcast(t, jnp.int32)
  e = ((bits >> 23) - 127).astype(jnp.float32)
  m = plsc.bitcast((bits & 0x007FFFFF) | 0x3F800000, jnp.float32)
  s = (m - 1.0) / (m + 1.0)
  s2 = s * s
  p = 2.0 / 9.0
  p = 2.0 / 7.0 + s2 * p
  p = 2.0 / 5.0 + s2 * p
  p = 2.0 / 3.0 + s2 * p
  return e * _LN2 + s * (2.0 + s2 * p)


def _lane_shift(v, idx):
  return v.at[idx].get(mode="promise_in_bounds")


def _sc_body(x_hbm, p_hbm, z_hbm, lj_hbm, pbuf, tab, xbuf0, xbuf1, zbuf0,
             zbuf1, ljbuf0, ljbuf1, sem_in0, sem_in1, sem_out0, sem_out1):
  wid = lax.axis_index("s") * _NC + lax.axis_index("c")
  base = wid * _PER_W
  xbufs = (xbuf0, xbuf1)
  zbufs = (zbuf0, zbuf1)
  ljbufs = (ljbuf0, ljbuf1)
  sems_in = (sem_in0, sem_in1)
  sems_out = (sem_out0, sem_out1)

  in_d = [None, None]
  in_d[0] = pltpu.async_copy(x_hbm.at[pl.ds(base, _CH)], xbufs[0],
                             sems_in[0])

  pltpu.sync_copy(p_hbm, pbuf)
  pv = pbuf[...]
  io = lax.iota(jnp.int32, 16)
  mask_w = io < _NUM_BINS
  mask_h = (io >= _NUM_BINS) & (io < 2 * _NUM_BINS)
  neg = jnp.float32(-3.4e38)

  mw = jnp.max(jnp.where(mask_w, pv, neg))
  ew = jnp.exp(pv - mw)
  sw = jnp.sum(jnp.where(mask_w, ew, 0.0))
  w_v = (ew * (2.0 * _TB)) / sw
  mh = jnp.max(jnp.where(mask_h, pv, neg))
  eh = jnp.exp(pv - mh)
  sh = jnp.sum(jnp.where(mask_h, eh, 0.0))
  h_v = (eh * (2.0 * _TB)) / sh
  d_v = jnp.maximum(pv, 0.0) + _vlog(1.0 + jnp.exp(-jnp.abs(pv))) + 1e-5

  cw = plsc.cumsum(jnp.where(mask_w, w_v, 0.0))
  ch = plsc.cumsum(jnp.where(mask_h, h_v, 0.0))

  cap = jnp.int32(15)
  x_k1 = cw - _TB
  x_k = jnp.where(io == 0, -_TB,
                  _lane_shift(cw, jnp.maximum(io - 1, 0)) - _TB)
  rw = 1.0 / (x_k1 - x_k + 1e-8)
  y_k = jnp.where(io == 0, -_TB,
                  _lane_shift(ch, jnp.minimum(io + 4, cap)) - _TB)
  y_k1 = _lane_shift(ch, jnp.minimum(io + 5, cap)) - _TB
  dy = y_k1 - y_k
  d_k = _lane_shift(d_v, jnp.minimum(io + 10, cap))
  d_k1 = _lane_shift(d_v, jnp.minimum(io + 11, cap))
  s_k = _lane_shift(h_v, jnp.minimum(io + 5, cap)) / w_v
  s8 = s_k + 1e-8
  mid = d_k + d_k1 - 2.0 * s_k
  dk8 = d_k + 1e-8
  h1 = s8 - d_k
  a1 = 2.0 * h1

  tab[0, :] = x_k
  tab[1, :] = rw
  tab[2, :] = y_k
  tab[3, :] = dy
  tab[4, :] = s8
  tab[5, :] = mid
  tab[6, :] = dk8
  tab[7, :] = h1
  tab[8, :] = a1

  k1 = jnp.sum(jnp.where(io == 0, x_k1, 0.0))
  k2 = jnp.sum(jnp.where(io == 1, x_k1, 0.0))
  k3 = jnp.sum(jnp.where(io == 2, x_k1, 0.0))
  k4 = jnp.sum(jnp.where(io == 3, x_k1, 0.0))

  q_idx = [jnp.full((16,), q, jnp.int32) for q in range(9)]

  def compute(xb, zb, ljb):
    @plsc.parallel_loop(0, _CH, step=_LANES, unroll=4)
    def _loop(off):
      sl = pl.ds(off, _LANES)
      xv = xb[sl]
      inside = jnp.abs(xv) <= _TB
      b = ((k1 < xv).astype(jnp.int32) + (k2 < xv).astype(jnp.int32)
           + (k3 < xv).astype(jnp.int32) + (k4 < xv).astype(jnp.int32))
      g_xk = plsc.load_gather(tab, [q_idx[0], b])
      g_rw = plsc.load_gather(tab, [q_idx[1], b])
      g_yk = plsc.load_gather(tab, [q_idx[2], b])
      g_dy = plsc.load_gather(tab, [q_idx[3], b])
      g_s8 = plsc.load_gather(tab, [q_idx[4], b])
      g_mid = plsc.load_gather(tab, [q_idx[5], b])
      g_dk8 = plsc.load_gather(tab, [q_idx[6], b])
      g_h1 = plsc.load_gather(tab, [q_idx[7], b])
      g_a1 = plsc.load_gather(tab, [q_idx[8], b])

      xi = jnp.clip((xv - g_xk) * g_rw, 0.0, 1.0)
      t = xi * (1.0 - xi)
      d8 = g_s8 + g_mid * t
      inv = 1.0 / d8
      numz = xi * (g_dk8 + g_h1 * xi)
      z_in = g_yk + g_dy * (numz * inv)
      numj = (g_mid * xi + g_a1) * xi + g_dk8
      r = g_s8 * inv
      lj_in = _vlog(numj * (r * r))
      zb[sl] = jnp.where(inside, z_in, xv)
      ljb[sl] = jnp.where(inside, lj_in, 0.0)

  out_d = [None, None]
  for g in range(_CHUNKS):
    b = g % 2
    off = base + g * _CH
    in_d[b].wait()
    if g + 1 < _CHUNKS:
      nb = (g + 1) % 2
      in_d[nb] = pltpu.async_copy(x_hbm.at[pl.ds(off + _CH, _CH)],
                                  xbufs[nb], sems_in[nb])
    if out_d[b] is not None:
      out_d[b][0].wait()
      out_d[b][1].wait()
    compute(xbufs[b], zbufs[b], ljbufs[b])
    out_d[b] = (
        pltpu.async_copy(zbufs[b], z_hbm.at[pl.ds(off, _CH)], sems_out[b]),
        pltpu.async_copy(ljbufs[b], lj_hbm.at[pl.ds(off, _CH)],
                         sems_out[b]),
    )
  out_d[0][0].wait()
  out_d[0][1].wait()
  out_d[1][0].wait()
  out_d[1][1].wait()


@jax.jit
def _run(x_flat, params):
  mesh = plsc.VectorSubcoreMesh(core_axis_name="c", subcore_axis_name="s",
                                num_cores=_NC, num_subcores=_NS)
  f = pl.kernel(
      _sc_body,
      out_type=[jax.ShapeDtypeStruct((_N,), jnp.float32),
                jax.ShapeDtypeStruct((_N,), jnp.float32)],
      mesh=mesh,
      compiler_params=pltpu.CompilerParams(needs_layout_passes=False),
      scratch_types=[
          pltpu.VMEM((16,), jnp.float32),
          pltpu.VMEM((9, 16), jnp.float32),
          pltpu.VMEM((_CH,), jnp.float32),
          pltpu.VMEM((_CH,), jnp.float32),
          pltpu.VMEM((_CH,), jnp.float32),
          pltpu.VMEM((_CH,), jnp.float32),
          pltpu.VMEM((_CH,), jnp.float32),
          pltpu.VMEM((_CH,), jnp.float32),
          pltpu.SemaphoreType.DMA,
          pltpu.SemaphoreType.DMA,
          pltpu.SemaphoreType.DMA,
          pltpu.SemaphoreType.DMA,
      ],
  )
  return f(x_flat, params)


def kernel(x, params):
  z, lj = _run(x[:, 0], params)
  return (z[:, None], lj)
